# baseline (speedup 1.0000x reference)
import functools

import jax
import jax.numpy as jnp
from jax import lax
from jax.experimental import pallas as pl
from jax.experimental.pallas import tpu as pltpu
from jax.experimental.pallas import tpu_sc as plsc

_B = 16384
_C = 784
_NB = 64
_NC = 208
_NN = 512
_KS = 16
_NOUT = _NB + _NC + _NN // _KS
_L = 16
_NCORES = 2
_NSUB = 16
_NW = _NCORES * _NSUB
_ROWS_PER_W = _B // _NW
_GROUPS = _ROWS_PER_W // _L
_VPC = _C // _L
_XROWS = _B * _C // 128
_PROWS = _B * _L // 128
_CHUNK = _L * _C // 128


def _sc_partial_rowsum(x128, a, cvec):
    mesh = plsc.VectorSubcoreMesh(core_axis_name="c", subcore_axis_name="s")

    @functools.partial(
        pl.kernel,
        mesh=mesh,
        out_type=jax.ShapeDtypeStruct((_B * _L,), jnp.float32),
        scratch_types=[
            pltpu.VMEM((_L * _C,), jnp.float32),
            pltpu.VMEM((_L * _C,), jnp.float32),
            pltpu.VMEM((_L * _L,), jnp.float32),
            pltpu.VMEM((_L * _L,), jnp.float32),
            pltpu.VMEM((_C,), jnp.float32),
            pltpu.VMEM((_L,), jnp.float32),
            pltpu.SemaphoreType.DMA,
            pltpu.SemaphoreType.DMA,
            pltpu.SemaphoreType.DMA,
            pltpu.SemaphoreType.DMA,
        ],
    )
    def k(x_hbm, a_hbm, c_hbm, p_hbm, buf0, buf1, pb0, pb1, a_v, c_v,
          isem0, isem1, osem0, osem1):
        wid = lax.axis_index("s") * _NCORES + lax.axis_index("c")
        base = wid * _ROWS_PER_W
        pltpu.sync_copy(a_hbm, a_v)
        pltpu.sync_copy(c_hbm, c_v)
        cv = c_v[...]

        bufs = (buf0, buf1)
        pbs = (pb0, pb1)
        isems = (isem0, isem1)
        osems = (osem0, osem1)

        def in_slice(g):
            row0 = base + g * _L
            return x_hbm.at[pl.ds(row0 * _C, _L * _C)]

        def out_slice(g):
            row0 = base + g * _L
            return p_hbm.at[pl.ds(row0 * _L, _L * _L)]

        pltpu.async_copy(in_slice(0), buf0, isem0)

        def step(i, carry):
            for s in range(2):
                g = 2 * i + s
                buf, pb = bufs[s], pbs[s]
                isem, osem = isems[s], osems[s]
                o = 1 - s

                @pl.when(g + 1 < _GROUPS)
                def _():
                    pltpu.async_copy(in_slice(g + 1), bufs[o], isems[o])

                pltpu.make_async_copy(in_slice(g), buf, isem).wait()

                @pl.when(i > 0)
                def _():
                    pltpu.make_async_copy(pb, out_slice(g), osem).wait()

                accs = [cv] * _L
                for j in range(_VPC):
                    aj = a_v[pl.ds(j * _L, _L)]
                    for rr in range(_L):
                        accs[rr] = accs[rr] + buf[pl.ds(rr * _C + j * _L, _L)] * aj
                for rr in range(_L):
                    pb[pl.ds(rr * _L, _L)] = accs[rr]
                pltpu.async_copy(pb, out_slice(g), osem)
            return carry

        lax.fori_loop(0, _GROUPS // 2, step, 0)
        pltpu.make_async_copy(pb0, out_slice(_GROUPS - 2), osem0).wait()
        pltpu.make_async_copy(pb1, out_slice(_GROUPS - 1), osem1).wait()

    return k(x128, a, cvec)


def _tc_final_rowsum(p, moh):
    def body(p_ref, m_ref, o_ref):
        o_ref[...] = jax.lax.dot(
            p_ref[...], m_ref[...],
            precision=jax.lax.Precision.HIGHEST,
            preferred_element_type=jnp.float32)

    return pl.pallas_call(
        body,
        out_shape=jax.ShapeDtypeStruct((_PROWS, 8), jnp.float32),
        grid=(8,),
        in_specs=[
            pl.BlockSpec((_PROWS // 8, 128), lambda i: (i, 0)),
            pl.BlockSpec((128, 8), lambda i: (0, 0)),
        ],
        out_specs=pl.BlockSpec((_PROWS // 8, 8), lambda i: (i, 0)),
    )(p, moh)


def kernel(input_linear, w1, b1, w2, b2, K, train_size, num_cat_variable,
           num_num_variable, num_bin_variable,
           vectorized_cate_col_name_num_list):
    x1d = input_linear.reshape(_B * _C)

    counts = jnp.asarray(vectorized_cate_col_name_num_list, dtype=jnp.int32)
    cum = jnp.cumsum(counts)
    jj = jnp.arange(_NC, dtype=jnp.int32)
    seg = jnp.sum((cum[None, :] <= jj[:, None]).astype(jnp.int32), axis=1)
    seg = jnp.minimum(seg, _NC - 1)
    gmap = jnp.concatenate([
        jnp.arange(_NB, dtype=jnp.int32),
        _NB + seg,
        _NB + _NC + jnp.arange(_NN, dtype=jnp.int32) // _KS,
    ])
    onehot = (gmap[:, None] == jnp.arange(_NOUT, dtype=jnp.int32)[None, :])
    w2g = jnp.dot(onehot.astype(jnp.float32), w2,
                  precision=jax.lax.Precision.HIGHEST)
    a = w1 * w2g
    cconst = jnp.dot(b1, w2g, precision=jax.lax.Precision.HIGHEST) + jnp.sum(b2)
    cvec = jnp.full((_L,), cconst / _L, dtype=jnp.float32)

    lane = jnp.arange(128, dtype=jnp.int32)
    moh = (lane[:, None] // _L == jnp.arange(8, dtype=jnp.int32)[None, :])
    moh = moh.astype(jnp.float32)

    p = _sc_partial_rowsum(x1d, a, cvec)
    out = _tc_final_rowsum(p.reshape(_PROWS, 128), moh)
    return out.reshape(_B, 1)


# speedup vs baseline: 1.4473x; 1.4473x over previous
import functools

import jax
import jax.numpy as jnp
from jax import lax
from jax.experimental import pallas as pl
from jax.experimental.pallas import tpu as pltpu
from jax.experimental.pallas import tpu_sc as plsc

_B = 16384
_C = 784
_NB = 64
_NC = 208
_NN = 512
_KS = 16
_NOUT = _NB + _NC + _NN // _KS
_L = 16
_NCORES = 2
_NSUB = 16
_NW = _NCORES * _NSUB
_ROWS_PER_W = _B // _NW
_GROUPS = _ROWS_PER_W // _L
_VPC = _C // _L
_XROWS = _B * _C // 128
_PROWS = _B * _L // 128
_CHUNK = _L * _C // 128


def _sc_partial_rowsum(x2d, a, cvec):
    mesh = plsc.VectorSubcoreMesh(core_axis_name="c", subcore_axis_name="s")

    @functools.partial(
        pl.kernel,
        mesh=mesh,
        out_type=jax.ShapeDtypeStruct((_B * _L,), jnp.float32),
        compiler_params=pltpu.CompilerParams(use_tc_tiling_on_sc=True),
        scratch_types=[
            pltpu.VMEM((_L, _C), jnp.float32),
            pltpu.VMEM((_L, _C), jnp.float32),
            pltpu.VMEM((_L * _L,), jnp.float32),
            pltpu.VMEM((_L * _L,), jnp.float32),
            pltpu.VMEM((_C,), jnp.float32),
            pltpu.VMEM((_L,), jnp.float32),
            pltpu.SemaphoreType.DMA,
            pltpu.SemaphoreType.DMA,
            pltpu.SemaphoreType.DMA,
            pltpu.SemaphoreType.DMA,
        ],
    )
    def k(x_hbm, a_hbm, c_hbm, p_hbm, buf0, buf1, pb0, pb1, a_v, c_v,
          isem0, isem1, osem0, osem1):
        wid = lax.axis_index("s") * _NCORES + lax.axis_index("c")
        base = wid * _ROWS_PER_W
        pltpu.sync_copy(a_hbm, a_v)
        pltpu.sync_copy(c_hbm, c_v)
        cv = c_v[...]

        bufs = (buf0, buf1)
        pbs = (pb0, pb1)
        isems = (isem0, isem1)
        osems = (osem0, osem1)

        def in_slice(g):
            row0 = base + g * _L
            return x_hbm.at[pl.ds(row0, _L), :]

        def out_slice(g):
            row0 = base + g * _L
            return p_hbm.at[pl.ds(row0 * _L, _L * _L)]

        pltpu.async_copy(in_slice(0), buf0, isem0)

        def step(i, carry):
            for s in range(2):
                g = 2 * i + s
                buf, pb = bufs[s], pbs[s]
                isem, osem = isems[s], osems[s]
                o = 1 - s

                @pl.when(g + 1 < _GROUPS)
                def _():
                    pltpu.async_copy(in_slice(g + 1), bufs[o], isems[o])

                pltpu.make_async_copy(in_slice(g), buf, isem).wait()

                @pl.when(i > 0)
                def _():
                    pltpu.make_async_copy(pb, out_slice(g), osem).wait()

                accs = [cv] * _L
                for j in range(_VPC):
                    aj = a_v[pl.ds(j * _L, _L)]
                    for rr in range(_L):
                        accs[rr] = accs[rr] + buf[rr, pl.ds(j * _L, _L)] * aj
                for rr in range(_L):
                    pb[pl.ds(rr * _L, _L)] = accs[rr]
                pltpu.async_copy(pb, out_slice(g), osem)
            return carry

        lax.fori_loop(0, _GROUPS // 2, step, 0)
        pltpu.make_async_copy(pb0, out_slice(_GROUPS - 2), osem0).wait()
        pltpu.make_async_copy(pb1, out_slice(_GROUPS - 1), osem1).wait()

    return k(x2d, a, cvec)


def _tc_final_rowsum(p, moh):
    def body(p_ref, m_ref, o_ref):
        o_ref[...] = jax.lax.dot(
            p_ref[...], m_ref[...],
            precision=jax.lax.Precision.HIGHEST,
            preferred_element_type=jnp.float32)

    return pl.pallas_call(
        body,
        out_shape=jax.ShapeDtypeStruct((_PROWS, 8), jnp.float32),
        grid=(8,),
        in_specs=[
            pl.BlockSpec((_PROWS // 8, 128), lambda i: (i, 0)),
            pl.BlockSpec((128, 8), lambda i: (0, 0)),
        ],
        out_specs=pl.BlockSpec((_PROWS // 8, 8), lambda i: (i, 0)),
    )(p, moh)


def kernel(input_linear, w1, b1, w2, b2, K, train_size, num_cat_variable,
           num_num_variable, num_bin_variable,
           vectorized_cate_col_name_num_list):
    x2d = input_linear.reshape(_B, _C)

    counts = jnp.asarray(vectorized_cate_col_name_num_list, dtype=jnp.int32)
    cum = jnp.cumsum(counts)
    jj = jnp.arange(_NC, dtype=jnp.int32)
    seg = jnp.sum((cum[None, :] <= jj[:, None]).astype(jnp.int32), axis=1)
    seg = jnp.minimum(seg, _NC - 1)
    gmap = jnp.concatenate([
        jnp.arange(_NB, dtype=jnp.int32),
        _NB + seg,
        _NB + _NC + jnp.arange(_NN, dtype=jnp.int32) // _KS,
    ])
    onehot = (gmap[:, None] == jnp.arange(_NOUT, dtype=jnp.int32)[None, :])
    w2g = jnp.dot(onehot.astype(jnp.float32), w2,
                  precision=jax.lax.Precision.HIGHEST)
    a = w1 * w2g
    cconst = jnp.dot(b1, w2g, precision=jax.lax.Precision.HIGHEST) + jnp.sum(b2)
    cvec = jnp.full((_L,), cconst / _L, dtype=jnp.float32)

    lane = jnp.arange(128, dtype=jnp.int32)
    moh = (lane[:, None] // _L == jnp.arange(8, dtype=jnp.int32)[None, :])
    moh = moh.astype(jnp.float32)

    p = _sc_partial_rowsum(x2d, a, cvec)
    out = _tc_final_rowsum(p.reshape(_PROWS, 128), moh)
    return out.reshape(_B, 1)


# speedup vs baseline: 2.1100x; 1.4579x over previous
import functools

import jax
import jax.numpy as jnp
from jax import lax
from jax.experimental import pallas as pl
from jax.experimental.pallas import tpu as pltpu
from jax.experimental.pallas import tpu_sc as plsc

_B = 16384
_C = 784
_NB = 64
_NC = 208
_NN = 512
_KS = 16
_NOUT = _NB + _NC + _NN // _KS
_L = 16
_NCORES = 2
_NSUB = 16
_NW = _NCORES * _NSUB
_ROWS_PER_W = _B // _NW
_GROUPS = _ROWS_PER_W // _L
_VPC = _C // _L
_XROWS = _B * _C // 128
_PROWS = _B * _L // 128
_CHUNK = _L * _C // 128


def _sc_partial_rowsum(x2d, a, cvec):
    mesh = plsc.VectorSubcoreMesh(core_axis_name="c", subcore_axis_name="s")

    @functools.partial(
        pl.kernel,
        mesh=mesh,
        out_type=jax.ShapeDtypeStruct((_B * _L,), jnp.float32),
        scratch_types=[
            pltpu.VMEM((_L, _C), jnp.float32),
            pltpu.VMEM((_L, _C), jnp.float32),
            pltpu.VMEM((_L * _L,), jnp.float32),
            pltpu.VMEM((_L * _L,), jnp.float32),
            pltpu.VMEM((_C,), jnp.float32),
            pltpu.VMEM((_L,), jnp.float32),
            pltpu.SemaphoreType.DMA,
            pltpu.SemaphoreType.DMA,
            pltpu.SemaphoreType.DMA,
            pltpu.SemaphoreType.DMA,
        ],
    )
    def k(x_hbm, a_hbm, c_hbm, p_hbm, buf0, buf1, pb0, pb1, a_v, c_v,
          isem0, isem1, osem0, osem1):
        wid = lax.axis_index("s") * _NCORES + lax.axis_index("c")
        base = wid * _ROWS_PER_W
        pltpu.sync_copy(a_hbm, a_v)
        pltpu.sync_copy(c_hbm, c_v)
        cv = c_v[...]

        bufs = (buf0, buf1)
        pbs = (pb0, pb1)
        isems = (isem0, isem1)
        osems = (osem0, osem1)

        def in_slice(g):
            row0 = base + g * _L
            return x_hbm.at[pl.ds(row0, _L), :]

        def out_slice(g):
            row0 = base + g * _L
            return p_hbm.at[pl.ds(row0 * _L, _L * _L)]

        pltpu.async_copy(in_slice(0), buf0, isem0)

        def step(i, carry):
            for s in range(2):
                g = 2 * i + s
                buf, pb = bufs[s], pbs[s]
                isem, osem = isems[s], osems[s]
                o = 1 - s

                @pl.when(g + 1 < _GROUPS)
                def _():
                    pltpu.async_copy(in_slice(g + 1), bufs[o], isems[o])

                pltpu.make_async_copy(in_slice(g), buf, isem).wait()

                @pl.when(i > 0)
                def _():
                    pltpu.make_async_copy(pb, out_slice(g), osem).wait()

                for half in range(2):
                    @plsc.parallel_loop(0, _VPC, 1, unroll=7,
                                        carry=(cv,) * (_L // 2))
                    def jloop(j, accs, half=half):
                        off = j * _L
                        aj = a_v[pl.ds(off, _L)]
                        return tuple(
                            accs[r]
                            + buf[half * (_L // 2) + r, pl.ds(off, _L)] * aj
                            for r in range(_L // 2))
                    accs = jloop
                    for r in range(_L // 2):
                        rr = half * (_L // 2) + r
                        pb[pl.ds(rr * _L, _L)] = accs[r]
                pltpu.async_copy(pb, out_slice(g), osem)
            return carry

        lax.fori_loop(0, _GROUPS // 2, step, 0)
        pltpu.make_async_copy(pb0, out_slice(_GROUPS - 2), osem0).wait()
        pltpu.make_async_copy(pb1, out_slice(_GROUPS - 1), osem1).wait()

    return k(x2d, a, cvec)


def _tc_final_rowsum(p, moh):
    def body(p_ref, m_ref, o_ref):
        o_ref[...] = jax.lax.dot(
            p_ref[...], m_ref[...],
            precision=jax.lax.Precision.HIGHEST,
            preferred_element_type=jnp.float32)

    return pl.pallas_call(
        body,
        out_shape=jax.ShapeDtypeStruct((_PROWS, 8), jnp.float32),
        grid=(8,),
        in_specs=[
            pl.BlockSpec((_PROWS // 8, 128), lambda i: (i, 0)),
            pl.BlockSpec((128, 8), lambda i: (0, 0)),
        ],
        out_specs=pl.BlockSpec((_PROWS // 8, 8), lambda i: (i, 0)),
    )(p, moh)


def kernel(input_linear, w1, b1, w2, b2, K, train_size, num_cat_variable,
           num_num_variable, num_bin_variable,
           vectorized_cate_col_name_num_list):
    x2d = input_linear.reshape(_B, _C)

    counts = jnp.asarray(vectorized_cate_col_name_num_list, dtype=jnp.int32)
    cum = jnp.cumsum(counts)
    jj = jnp.arange(_NC, dtype=jnp.int32)
    seg = jnp.sum((cum[None, :] <= jj[:, None]).astype(jnp.int32), axis=1)
    seg = jnp.minimum(seg, _NC - 1)
    gmap = jnp.concatenate([
        jnp.arange(_NB, dtype=jnp.int32),
        _NB + seg,
        _NB + _NC + jnp.arange(_NN, dtype=jnp.int32) // _KS,
    ])
    onehot = (gmap[:, None] == jnp.arange(_NOUT, dtype=jnp.int32)[None, :])
    w2g = jnp.dot(onehot.astype(jnp.float32), w2,
                  precision=jax.lax.Precision.HIGHEST)
    a = w1 * w2g
    cconst = jnp.dot(b1, w2g, precision=jax.lax.Precision.HIGHEST) + jnp.sum(b2)
    cvec = jnp.full((_L,), cconst / _L, dtype=jnp.float32)

    lane = jnp.arange(128, dtype=jnp.int32)
    moh = (lane[:, None] // _L == jnp.arange(8, dtype=jnp.int32)[None, :])
    moh = moh.astype(jnp.float32)

    p = _sc_partial_rowsum(x2d, a, cvec)
    out = _tc_final_rowsum(p.reshape(_PROWS, 128), moh)
    return out.reshape(_B, 1)
